# Initial kernel scaffold; baseline (speedup 1.0000x reference)
import jax
import jax.numpy as jnp
from jax.experimental import pallas as pl


def kernel(cls_pred, loc_pred, cls_true, loc_true, reg_mask):
    raise NotImplementedError("write your pallas kernel here")


# speedup vs baseline: 2.6600x; 2.6600x over previous
import jax
import jax.numpy as jnp
from jax import lax
from jax.experimental import pallas as pl
from jax.experimental.pallas import tpu as pltpu

B, W, H, C, M = 4, 64, 64, 20, 100
WH = W * H
CH = 512
NCH = WH // CH
BIG_I = 1 << 30


def _matcher_body(cls_pred_ref, loc_pred_ref, cls_true_ref, loc_true_ref,
                  argmin_ref, clsid_ref, bv_ref, bi_ref):
    ch = pl.program_id(1)
    pt = cls_pred_ref[0]
    ct = jnp.equal(cls_true_ref[0], 1).astype(jnp.float32)

    neg = 0.75 * (pt * pt) * (-jnp.log(1.0 - pt))
    pos = 0.25 * ((1.0 - pt) * (1.0 - pt)) * (-jnp.log(pt + 1e-08))
    d = pos - neg

    cls_loss = ct[:, 0][:, None] * d[0, :][None, :]
    for c in range(1, C):
        cls_loss = cls_loss + ct[:, c][:, None] * d[c, :][None, :]

    lp = loc_pred_ref[0] / jnp.float32(W)
    lt = loc_true_ref[0]

    b1_ymin = lp[0, :][None, :]
    b1_xmin = lp[1, :][None, :]
    b1_ymax = lp[2, :][None, :]
    b1_xmax = lp[3, :][None, :]
    b2_ymin = lt[:, 0][:, None]
    b2_xmin = lt[:, 1][:, None]
    b2_ymax = lt[:, 2][:, None]
    b2_xmax = lt[:, 3][:, None]

    zero = jnp.float32(0.0)
    b1_area = (jnp.maximum(zero, b1_xmax - b1_xmin)
               * jnp.maximum(zero, b1_ymax - b1_ymin))
    b2_area = (jnp.maximum(zero, b2_xmax - b2_xmin)
               * jnp.maximum(zero, b2_ymax - b2_ymin))

    i_w = jnp.maximum(zero, jnp.minimum(b1_xmax, b2_xmax)
                      - jnp.maximum(b1_xmin, b2_xmin))
    i_h = jnp.maximum(zero, jnp.minimum(b1_ymax, b2_ymax)
                      - jnp.maximum(b1_ymin, b2_ymin))
    i_area = i_w * i_h
    union = b1_area + b2_area - i_area
    iou = jnp.where(union > 0, i_area / jnp.where(union > 0, union, 1.0), 0.0)

    e_w = jnp.maximum(zero, jnp.maximum(b1_xmax, b2_xmax)
                      - jnp.minimum(b1_xmin, b2_xmin))
    e_h = jnp.maximum(zero, jnp.maximum(b1_ymax, b2_ymax)
                      - jnp.minimum(b1_ymin, b2_ymin))
    e_area = e_w * e_h
    giou = iou - jnp.where(e_area > 0,
                           (e_area - union) / jnp.where(e_area > 0, e_area, 1.0),
                           0.0)
    giou_l = 1.0 - giou

    reg = (jnp.abs(b2_ymin - b1_ymin) + jnp.abs(b2_xmin - b1_xmin)
           + jnp.abs(b2_ymax - b1_ymax) + jnp.abs(b2_xmax - b1_xmax))

    total = 2.0 * cls_loss + 5.0 * reg + 2.0 * giou_l

    iota = lax.broadcasted_iota(jnp.int32, (M, CH), 1) + ch * CH
    cmin = jnp.min(total, axis=1)
    cam = jnp.min(jnp.where(total == cmin[:, None], iota, BIG_I), axis=1)

    @pl.when(ch == 0)
    def _():
        bv_ref[0, :] = cmin
        bi_ref[0, :] = cam

    @pl.when(ch != 0)
    def _():
        bv = bv_ref[0, :]
        upd = cmin < bv
        bi_ref[0, :] = jnp.where(upd, cam, bi_ref[0, :])
        bv_ref[0, :] = jnp.minimum(cmin, bv)

    argmin_ref[0, 0, :] = bi_ref[0, :]

    maxv = jnp.max(ct, axis=1, keepdims=True)
    iota_c = lax.broadcasted_iota(jnp.int32, (M, C), 1)
    clsid_ref[0, 0, :] = jnp.min(jnp.where(ct == maxv, iota_c, C), axis=1)


def kernel(cls_pred, loc_pred, cls_true, loc_true, reg_mask):
    del reg_mask
    cls_pred_t = cls_pred.reshape(B, WH, C).transpose(0, 2, 1)
    loc_pred_t = loc_pred.reshape(B, WH, 4).transpose(0, 2, 1)

    am, cid = pl.pallas_call(
        _matcher_body,
        grid=(B, NCH),
        in_specs=[
            pl.BlockSpec((1, C, CH), lambda b, ch: (b, 0, ch)),
            pl.BlockSpec((1, 4, CH), lambda b, ch: (b, 0, ch)),
            pl.BlockSpec((1, M, C), lambda b, ch: (b, 0, 0)),
            pl.BlockSpec((1, M, 4), lambda b, ch: (b, 0, 0)),
        ],
        out_specs=[
            pl.BlockSpec((1, 1, M), lambda b, ch: (b, 0, 0)),
            pl.BlockSpec((1, 1, M), lambda b, ch: (b, 0, 0)),
        ],
        out_shape=[
            jax.ShapeDtypeStruct((B, 1, M), jnp.int32),
            jax.ShapeDtypeStruct((B, 1, M), jnp.int32),
        ],
        scratch_shapes=[
            pltpu.VMEM((1, M), jnp.float32),
            pltpu.VMEM((1, M), jnp.int32),
        ],
    )(cls_pred_t, loc_pred_t, cls_true, loc_true)

    am = am.reshape(B, M)[..., None]
    cid = cid.reshape(B, M)[..., None]
    batch = jnp.tile(jnp.arange(B, dtype=jnp.int32)[:, None], (1, M))[..., None]
    return jnp.concatenate((batch, am, cid), axis=-1)


# speedup vs baseline: 3.9541x; 1.4865x over previous
import jax
import jax.numpy as jnp
from jax import lax
from jax.experimental import pallas as pl

B, W, H, C, M = 4, 64, 64, 20, 100
WH = W * H
MT = 8
BIG_I = 1 << 30


def _matcher_body(cls_pred_ref, loc_pred_ref, cls_true_ref, loc_true_ref,
                  argmin_ref, clsid_ref):
    pt = cls_pred_ref[0]
    ct = jnp.equal(cls_true_ref[0], 1).astype(jnp.float32)

    neg = 0.75 * (pt * pt) * (-jnp.log(1.0 - pt))
    pos = 0.25 * ((1.0 - pt) * (1.0 - pt)) * (-jnp.log(pt + 1e-08))
    d = pos - neg

    parts = []
    for mt in range(0, M, MT):
        ct_t = ct[mt:mt + MT]
        acc = ct_t[:, 0][:, None] * d[0, :][None, :]
        for c in range(1, C):
            acc = acc + ct_t[:, c][:, None] * d[c, :][None, :]
        parts.append(acc)
    cls_loss = jnp.concatenate(parts, axis=0)

    lp = loc_pred_ref[0] / jnp.float32(W)
    lt = loc_true_ref[0]

    b1_ymin = lp[0, :][None, :]
    b1_xmin = lp[1, :][None, :]
    b1_ymax = lp[2, :][None, :]
    b1_xmax = lp[3, :][None, :]
    b2_ymin = lt[:, 0][:, None]
    b2_xmin = lt[:, 1][:, None]
    b2_ymax = lt[:, 2][:, None]
    b2_xmax = lt[:, 3][:, None]

    zero = jnp.float32(0.0)
    b1_area = (jnp.maximum(zero, b1_xmax - b1_xmin)
               * jnp.maximum(zero, b1_ymax - b1_ymin))
    b2_area = (jnp.maximum(zero, b2_xmax - b2_xmin)
               * jnp.maximum(zero, b2_ymax - b2_ymin))

    i_w = jnp.maximum(zero, jnp.minimum(b1_xmax, b2_xmax)
                      - jnp.maximum(b1_xmin, b2_xmin))
    i_h = jnp.maximum(zero, jnp.minimum(b1_ymax, b2_ymax)
                      - jnp.maximum(b1_ymin, b2_ymin))
    i_area = i_w * i_h
    union = b1_area + b2_area - i_area
    iou = jnp.where(union > 0, i_area / jnp.where(union > 0, union, 1.0), 0.0)

    e_w = jnp.maximum(zero, jnp.maximum(b1_xmax, b2_xmax)
                      - jnp.minimum(b1_xmin, b2_xmin))
    e_h = jnp.maximum(zero, jnp.maximum(b1_ymax, b2_ymax)
                      - jnp.minimum(b1_ymin, b2_ymin))
    e_area = e_w * e_h
    giou = iou - jnp.where(e_area > 0,
                           (e_area - union) / jnp.where(e_area > 0, e_area, 1.0),
                           0.0)
    giou_l = 1.0 - giou

    reg = (jnp.abs(b2_ymin - b1_ymin) + jnp.abs(b2_xmin - b1_xmin)
           + jnp.abs(b2_ymax - b1_ymax) + jnp.abs(b2_xmax - b1_xmax))

    total = 2.0 * cls_loss + 5.0 * reg + 2.0 * giou_l

    iota = lax.broadcasted_iota(jnp.int32, (M, WH), 1)
    cmin = jnp.min(total, axis=1)
    am = jnp.min(jnp.where(total == cmin[:, None], iota, BIG_I), axis=1)
    argmin_ref[0, 0, :] = am

    maxv = jnp.max(ct, axis=1, keepdims=True)
    iota_c = lax.broadcasted_iota(jnp.int32, (M, C), 1)
    clsid_ref[0, 0, :] = jnp.min(jnp.where(ct == maxv, iota_c, C), axis=1)


def kernel(cls_pred, loc_pred, cls_true, loc_true, reg_mask):
    del reg_mask
    cls_pred_t = cls_pred.reshape(B, WH, C).transpose(0, 2, 1)
    loc_pred_t = loc_pred.reshape(B, WH, 4).transpose(0, 2, 1)

    am, cid = pl.pallas_call(
        _matcher_body,
        grid=(B,),
        in_specs=[
            pl.BlockSpec((1, C, WH), lambda b: (b, 0, 0)),
            pl.BlockSpec((1, 4, WH), lambda b: (b, 0, 0)),
            pl.BlockSpec((1, M, C), lambda b: (b, 0, 0)),
            pl.BlockSpec((1, M, 4), lambda b: (b, 0, 0)),
        ],
        out_specs=[
            pl.BlockSpec((1, 1, M), lambda b: (b, 0, 0)),
            pl.BlockSpec((1, 1, M), lambda b: (b, 0, 0)),
        ],
        out_shape=[
            jax.ShapeDtypeStruct((B, 1, M), jnp.int32),
            jax.ShapeDtypeStruct((B, 1, M), jnp.int32),
        ],
    )(cls_pred_t, loc_pred_t, cls_true, loc_true)

    am = am.reshape(B, M)[..., None]
    cid = cid.reshape(B, M)[..., None]
    batch = jnp.tile(jnp.arange(B, dtype=jnp.int32)[:, None], (1, M))[..., None]
    return jnp.concatenate((batch, am, cid), axis=-1)
